# baseline (speedup 1.0000x reference)
import functools

import jax
import jax.numpy as jnp
from jax import lax
from jax.experimental import pallas as pl
from jax.experimental.pallas import tpu as pltpu
from jax.experimental.pallas import tpu_sc as plsc

_B = 16384
_VOCAB = 1_000_000
_NS = 26
_NH = 50
_ND = 13
_NF = _NS + _NH
_L = 16

_NC = 2
_NSUB = 16
_NW = _NC * _NSUB
_BPW = _B // _NW
_GPW = _BPW // _L
_NIDX = _NF * _BPW
_CHUNK = 128
_NCHUNK = _NIDX // _CHUNK
_FIRE = 8
_VOFF = _NS * _VOCAB


def _body(sp_hbm, vl_hbm, dn_hbm, tab_hbm, wb_hbm, out_hbm,
          raw_sp, raw_vl, dense, wb, idxs, vals, accs, sem):
    wid = lax.axis_index("s") * _NC + lax.axis_index("c")
    base = wid * _BPW
    pltpu.sync_copy(sp_hbm.at[pl.ds(base * _NS, _BPW * _NS)], raw_sp)
    pltpu.sync_copy(vl_hbm.at[pl.ds(base * _NH, _BPW * _NH)], raw_vl)
    pltpu.sync_copy(dn_hbm.at[pl.ds(base * _ND, _BPW * _ND)], dense)
    pltpu.sync_copy(wb_hbm, wb)

    lane = lax.iota(jnp.int32, _L)
    lane_ns = lane * _NS
    lane_nh = lane * _NH
    lane_nd = lane * _ND

    def build(g, c):
        for f in range(_NS):
            v = plsc.load_gather(raw_sp, [lane_ns + (g * _L * _NS + f)])
            idxs[pl.ds(f * _BPW + g * _L, _L)] = v + f * _VOCAB
        for h in range(_NH):
            v = plsc.load_gather(raw_vl, [lane_nh + (g * _L * _NH + h)])
            idxs[pl.ds((_NS + h) * _BPW + g * _L, _L)] = v + _VOFF
        return c

    lax.fori_loop(0, _GPW, build, 0)

    def fire(t, c):
        for j in range(_FIRE):
            r = t * _FIRE + j
            pltpu.make_async_copy(
                tab_hbm.at[idxs.at[pl.ds(r * _CHUNK, _CHUNK)]],
                vals.at[pl.ds(r * _CHUNK, _CHUNK)],
                sem).start()
        for j in range(_FIRE):
            r = t * _FIRE + j
            pltpu.make_async_copy(
                tab_hbm.at[idxs.at[pl.ds(r * _CHUNK, _CHUNK)]],
                vals.at[pl.ds(r * _CHUNK, _CHUNK)],
                sem).wait()
        return c

    lax.fori_loop(0, _NCHUNK // _FIRE, fire, 0)

    def reduce(g, c):
        off = g * _L
        acc = jnp.zeros((_L,), jnp.float32)
        for f in range(_NS):
            acc = acc + vals[pl.ds(f * _BPW + off, _L)]
        vsum = jnp.zeros((_L,), jnp.float32)
        cnt = jnp.zeros((_L,), jnp.float32)
        for h in range(_NH):
            p = (_NS + h) * _BPW + off
            v = vals[pl.ds(p, _L)]
            ix = idxs[pl.ds(p, _L)]
            m = ix > _VOFF
            vsum = vsum + jnp.where(m, v, 0.0)
            cnt = cnt + jnp.where(m, 1.0, 0.0)
        acc = acc + vsum / jnp.maximum(cnt, 1.0)
        for d in range(_ND):
            dv = plsc.load_gather(dense, [lane_nd + (off * _ND + d)])
            acc = acc + dv * wb[pl.ds(d * _L, _L)]
        accs[pl.ds(off, _L)] = acc
        return c

    lax.fori_loop(0, _GPW, reduce, 0)

    pltpu.sync_copy(accs, out_hbm.at[pl.ds(base, _BPW)])


@jax.jit
def _run(sparse_idx, dense_vals, varlen_idx, emb_flat, w_b):
    mesh = plsc.VectorSubcoreMesh(core_axis_name="c", subcore_axis_name="s")
    kfn = functools.partial(
        pl.kernel,
        out_type=jax.ShapeDtypeStruct((_B,), jnp.float32),
        mesh=mesh,
        compiler_params=pltpu.CompilerParams(needs_layout_passes=False),
        scratch_types=[
            pltpu.VMEM((_BPW * _NS,), jnp.int32),
            pltpu.VMEM((_BPW * _NH,), jnp.int32),
            pltpu.VMEM((_BPW * _ND,), jnp.float32),
            pltpu.VMEM((_ND * _L,), jnp.float32),
            pltpu.VMEM((_NIDX,), jnp.int32),
            pltpu.VMEM((_NIDX,), jnp.float32),
            pltpu.VMEM((_BPW,), jnp.float32),
            pltpu.SemaphoreType.DMA,
        ],
    )(_body)
    return kfn(sparse_idx, varlen_idx, dense_vals, emb_flat, w_b)


def kernel(sparse_idx, dense_vals, varlen_idx, emb_tables, W):
    emb_flat = emb_tables.reshape(-1)
    w_b = jnp.broadcast_to(W.reshape(_ND, 1), (_ND, _L)).reshape(-1)
    out = _run(sparse_idx.reshape(-1), dense_vals.reshape(-1),
               varlen_idx.reshape(-1), emb_flat, w_b)
    return out.reshape(_B, 1)


# speedup vs baseline: 1.7390x; 1.7390x over previous
import functools

import jax
import jax.numpy as jnp
from jax import lax
from jax.experimental import pallas as pl
from jax.experimental.pallas import tpu as pltpu
from jax.experimental.pallas import tpu_sc as plsc

_B = 16384
_VOCAB = 1_000_000
_NS = 26
_NH = 50
_ND = 13
_NF = _NS + _NH
_L = 16

_NC = 2
_NSUB = 16
_NW = _NC * _NSUB
_BPW = _B // _NW
_GPW = _BPW // _L
_NIDX = _NF * _BPW
_CHUNK = 128
_NCHUNK = _NIDX // _CHUNK
_FIRE = 8
_VOFF = _NS * _VOCAB


def _body(sp_hbm, vl_hbm, dn_hbm, tab_hbm, wb_hbm, out_hbm,
          raw_sp, raw_vl, dense, wb, idxs, vals, accs, sem):
    wid = lax.axis_index("s") * _NC + lax.axis_index("c")
    base = wid * _BPW
    pltpu.sync_copy(sp_hbm.at[pl.ds(base * _NS, _BPW * _NS)], raw_sp)
    pltpu.sync_copy(vl_hbm.at[pl.ds(base * _NH, _BPW * _NH)], raw_vl)
    pltpu.sync_copy(dn_hbm.at[pl.ds(base * _ND, _BPW * _ND)], dense)
    pltpu.sync_copy(wb_hbm, wb)

    lane = lax.iota(jnp.int32, _L)
    lane_ns = lane * _NS
    lane_nh = lane * _NH
    lane_nd = lane * _ND

    def build(g, c):
        for f in range(_NS):
            v = plsc.load_gather(raw_sp, [lane_ns + (g * _L * _NS + f)])
            idxs[pl.ds(f * _BPW + g * _L, _L)] = v + f * _VOCAB
        for h in range(_NH):
            v = plsc.load_gather(raw_vl, [lane_nh + (g * _L * _NH + h)])
            idxs[pl.ds((_NS + h) * _BPW + g * _L, _L)] = v + _VOFF
        return c

    lax.fori_loop(0, _GPW, build, 0)

    def fire(t, c):
        for j in range(_FIRE):
            r = t * _FIRE + j
            pltpu.make_async_copy(
                tab_hbm.at[idxs.at[pl.ds(r * _CHUNK, _CHUNK)]],
                vals.at[pl.ds(r * _CHUNK, _CHUNK)],
                sem).start()
        for j in range(_FIRE):
            r = t * _FIRE + j
            pltpu.make_async_copy(
                tab_hbm.at[idxs.at[pl.ds(r * _CHUNK, _CHUNK)]],
                vals.at[pl.ds(r * _CHUNK, _CHUNK)],
                sem).wait()
        return c

    lax.fori_loop(0, _NCHUNK // _FIRE, fire, 0)

    def reduce(g, c):
        off = g * _L
        acc = jnp.zeros((_L,), jnp.float32)
        for f in range(_NS):
            acc = acc + vals[pl.ds(f * _BPW + off, _L)]
        vsum = jnp.zeros((_L,), jnp.float32)
        cnt = jnp.zeros((_L,), jnp.float32)
        for h in range(_NH):
            p = (_NS + h) * _BPW + off
            v = vals[pl.ds(p, _L)]
            ix = idxs[pl.ds(p, _L)]
            m = ix > _VOFF
            vsum = vsum + jnp.where(m, v, 0.0)
            cnt = cnt + jnp.where(m, 1.0, 0.0)
        acc = acc + vsum / jnp.maximum(cnt, 1.0)
        for d in range(_ND):
            dv = plsc.load_gather(dense, [lane_nd + (off * _ND + d)])
            acc = acc + dv * wb[pl.ds(d * _L, _L)]
        accs[pl.ds(off, _L)] = acc
        return c

    lax.fori_loop(0, _GPW, reduce, 0)

    pltpu.sync_copy(accs, out_hbm.at[pl.ds(base, _BPW)])


@jax.jit
def _run(sparse_idx, dense_vals, varlen_idx, emb_flat, w_b):
    mesh = plsc.VectorSubcoreMesh(core_axis_name="c", subcore_axis_name="s")
    kfn = functools.partial(
        pl.kernel,
        out_type=jax.ShapeDtypeStruct((_B,), jnp.float32),
        mesh=mesh,
        compiler_params=pltpu.CompilerParams(needs_layout_passes=False),
        scratch_types=[
            pltpu.VMEM((_BPW * _NS,), jnp.int32),
            pltpu.VMEM((_BPW * _NH,), jnp.int32),
            pltpu.VMEM((_BPW * _ND,), jnp.float32),
            pltpu.VMEM((_ND * _L,), jnp.float32),
            pltpu.VMEM((_NIDX,), jnp.int32),
            pltpu.VMEM((_NIDX,), jnp.float32),
            pltpu.VMEM((_BPW,), jnp.float32),
            pltpu.SemaphoreType.DMA,
        ],
    )(_body)
    return kfn(sparse_idx, varlen_idx, dense_vals, emb_flat, w_b)


def kernel(sparse_idx, dense_vals, varlen_idx, emb_tables, W):
    emb_flat = jnp.concatenate(
        [emb_tables[f, :, 0] for f in range(_NS + 1)])
    w_b = jnp.broadcast_to(W.reshape(_ND, 1), (_ND, _L)).reshape(-1)
    out = _run(sparse_idx.reshape(-1), dense_vals.reshape(-1),
               varlen_idx.reshape(-1), emb_flat, w_b)
    return out.reshape(_B, 1)


# speedup vs baseline: 5.0391x; 2.8977x over previous
import functools

import jax
import jax.numpy as jnp
from jax import lax
from jax.experimental import pallas as pl
from jax.experimental.pallas import tpu as pltpu
from jax.experimental.pallas import tpu_sc as plsc

_B = 16384
_VOCAB = 1_000_000
_NS = 26
_NH = 50
_ND = 13
_NF = _NS + _NH
_NT = _NS + 1
_L = 16

_NC = 2
_NSUB = 16
_NW = _NC * _NSUB
_BPW = _B // _NW
_GPW = _BPW // _L
_NIDX = _NF * _BPW
_CHUNK = 128
_CPF = _BPW // _CHUNK
_NCHUNK = _NIDX // _CHUNK


def _body(*refs):
    sp_hbm, vl_hbm, dn_hbm, wb_hbm = refs[:4]
    tabs = refs[4:4 + _NT]
    out_hbm = refs[4 + _NT]
    idxs, vals, dense, wb, accs, ssem, gsem = refs[5 + _NT:]

    wid = lax.axis_index("s") * _NC + lax.axis_index("c")
    base = wid * _BPW

    def sp_stage(f, c):
        pltpu.make_async_copy(
            sp_hbm.at[pl.ds(f * _B + base, _BPW)],
            idxs.at[pl.ds(f * _BPW, _BPW)], ssem).start()
        return c

    def vl_stage(h, c):
        pltpu.make_async_copy(
            vl_hbm.at[pl.ds(h * _B + base, _BPW)],
            idxs.at[pl.ds((_NS + h) * _BPW, _BPW)], ssem).start()
        return c

    def dn_stage(d, c):
        pltpu.make_async_copy(
            dn_hbm.at[pl.ds(d * _B + base, _BPW)],
            dense.at[pl.ds(d * _BPW, _BPW)], ssem).start()
        return c

    lax.fori_loop(0, _NS, sp_stage, 0)
    lax.fori_loop(0, _NH, vl_stage, 0)
    lax.fori_loop(0, _ND, dn_stage, 0)
    pltpu.sync_copy(wb_hbm, wb)

    def stage_drain(f, c):
        pltpu.make_async_copy(
            sp_hbm.at[pl.ds(base, _BPW)],
            idxs.at[pl.ds(0, _BPW)], ssem).wait()
        return c

    lax.fori_loop(0, _NF + _ND, stage_drain, 0)

    for f in range(_NS):
        for j in range(_CPF):
            r = f * _CPF + j
            pltpu.make_async_copy(
                tabs[f].at[idxs.at[pl.ds(r * _CHUNK, _CHUNK)]],
                vals.at[pl.ds(r * _CHUNK, _CHUNK)], gsem).start()

    def vfire(t, c):
        r = _NS * _CPF + t
        pltpu.make_async_copy(
            tabs[_NS].at[idxs.at[pl.ds(r * _CHUNK, _CHUNK)]],
            vals.at[pl.ds(r * _CHUNK, _CHUNK)], gsem).start()
        return c

    lax.fori_loop(0, _NH * _CPF, vfire, 0)

    def gdrain(r, c):
        pltpu.make_async_copy(
            tabs[0].at[idxs.at[pl.ds(0, _CHUNK)]],
            vals.at[pl.ds(0, _CHUNK)], gsem).wait()
        return c

    lax.fori_loop(0, _NCHUNK, gdrain, 0)

    wd = [wb[pl.ds(d * _L, _L)] for d in range(_ND)]

    def reduce(g, c):
        off = g * _L
        acc = jnp.zeros((_L,), jnp.float32)
        for f in range(_NS):
            acc = acc + vals[pl.ds(f * _BPW + off, _L)]
        vsum = jnp.zeros((_L,), jnp.float32)
        cnt = jnp.zeros((_L,), jnp.float32)
        for h in range(_NH):
            p = (_NS + h) * _BPW + off
            v = vals[pl.ds(p, _L)]
            ix = idxs[pl.ds(p, _L)]
            m = ix > 0
            vsum = vsum + jnp.where(m, v, 0.0)
            cnt = cnt + jnp.where(m, 1.0, 0.0)
        acc = acc + vsum / jnp.maximum(cnt, 1.0)
        for d in range(_ND):
            acc = acc + dense[pl.ds(d * _BPW + off, _L)] * wd[d]
        accs[pl.ds(off, _L)] = acc
        return c

    lax.fori_loop(0, _GPW, reduce, 0)

    pltpu.sync_copy(accs, out_hbm.at[pl.ds(base, _BPW)])


@jax.jit
def _run(sp_t, dn_t, vl_t, w_b, *tabs):
    mesh = plsc.VectorSubcoreMesh(core_axis_name="c", subcore_axis_name="s")
    kfn = functools.partial(
        pl.kernel,
        out_type=jax.ShapeDtypeStruct((_B,), jnp.float32),
        mesh=mesh,
        compiler_params=pltpu.CompilerParams(
            needs_layout_passes=False, use_tc_tiling_on_sc=False),
        scratch_types=[
            pltpu.VMEM((_NIDX,), jnp.int32),
            pltpu.VMEM((_NIDX,), jnp.float32),
            pltpu.VMEM((_BPW * _ND,), jnp.float32),
            pltpu.VMEM((_ND * _L,), jnp.float32),
            pltpu.VMEM((_BPW,), jnp.float32),
            pltpu.SemaphoreType.DMA,
            pltpu.SemaphoreType.DMA,
        ],
    )(_body)
    return kfn(sp_t, vl_t, dn_t, w_b, *tabs)


def kernel(sparse_idx, dense_vals, varlen_idx, emb_tables, W):
    sp_t = sparse_idx.T.reshape(-1)
    vl_t = varlen_idx.T.reshape(-1)
    dn_t = dense_vals.T.reshape(-1)
    tabs = [emb_tables[f, :, 0] for f in range(_NT)]
    w_b = jnp.broadcast_to(W.reshape(_ND, 1), (_ND, _L)).reshape(-1)
    out = _run(sp_t, dn_t, vl_t, w_b, *tabs)
    return out.reshape(_B, 1)
